# Initial kernel scaffold; baseline (speedup 1.0000x reference)
import jax
import jax.numpy as jnp
from jax.experimental import pallas as pl


def kernel(x, gWih0, gWhh0, gbih0, gbhh0, gWih1, gWhh1, gbih1, gbhh1, attW1, attb1, attW2, attb2, projW1, projb1, ln_g, ln_b, projW2, projb2, dw1, dwb1, pw1, pwb1, bn1g, bn1b, dw2, dwb2, pw2, pwb2, bn2g, bn2b, gcW, gcb, gateW1, gateb1, gateW2, gateb2, temp):
    raise NotImplementedError("write your pallas kernel here")


# speedup vs baseline: 7.3087x; 7.3087x over previous
import jax
import jax.numpy as jnp
from jax.experimental import pallas as pl
from jax.experimental.pallas import tpu as pltpu

_B, _S, _C = 64, 64, 64
_H, _NE, _RH = 128, 4, 64
_F32 = jnp.float32


def _gelu(x):
    return 0.5 * x * (1.0 + jax.lax.erf(x * 0.7071067811865476))


def _gru_gate(gi, gh, h):
    r = jax.nn.sigmoid(gi[:, :_H] + gh[:, :_H])
    z = jax.nn.sigmoid(gi[:, _H:2 * _H] + gh[:, _H:2 * _H])
    n = jnp.tanh(gi[:, 2 * _H:] + r * gh[:, 2 * _H:])
    return (1.0 - z) * n + z * h


def _expert_body(xs_ref, wih0_ref, whh0_ref, bih0_ref, bhh0_ref,
                 wih1_ref, whh1_ref, bih1_ref, bhh1_ref,
                 aW1_ref, ab1_ref, aW2_ref, ab2_ref,
                 pW1_ref, pb1_ref, lng_ref, lnb_ref, pW2_ref, pb2_ref,
                 eo_ref, ysf_ref, ysb_ref, h2f_ref, h2b_ref):
    f32 = _F32
    wih0f, wih0b = wih0_ref[0, 0], wih0_ref[0, 1]
    whh0f, whh0b = whh0_ref[0, 0], whh0_ref[0, 1]
    bih0f, bih0b = bih0_ref[0, 0], bih0_ref[0, 1]
    bhh0f, bhh0b = bhh0_ref[0, 0], bhh0_ref[0, 1]

    def l0_step(t, hs):
        hf, hb = hs
        xt = xs_ref[t]
        xb = xs_ref[_S - 1 - t]
        gif = jnp.dot(xt, wih0f, preferred_element_type=f32) + bih0f
        ghf = jnp.dot(hf, whh0f, preferred_element_type=f32) + bhh0f
        hf = _gru_gate(gif, ghf, hf)
        gib = jnp.dot(xb, wih0b, preferred_element_type=f32) + bih0b
        ghb = jnp.dot(hb, whh0b, preferred_element_type=f32) + bhh0b
        hb = _gru_gate(gib, ghb, hb)
        ysf_ref[t] = hf
        ysb_ref[_S - 1 - t] = hb
        return (hf, hb)

    h0 = jnp.zeros((_B, _H), f32)
    jax.lax.fori_loop(0, _S, l0_step, (h0, h0))

    wih1f, wih1b = wih1_ref[0, 0], wih1_ref[0, 1]
    whh1f, whh1b = whh1_ref[0, 0], whh1_ref[0, 1]
    bih1f, bih1b = bih1_ref[0, 0], bih1_ref[0, 1]
    bhh1f, bhh1b = bhh1_ref[0, 0], bhh1_ref[0, 1]

    def l1_step(t, hs):
        hf, hb = hs
        uf = jnp.concatenate([ysf_ref[t], ysb_ref[t]], axis=-1)
        tb = _S - 1 - t
        ub = jnp.concatenate([ysf_ref[tb], ysb_ref[tb]], axis=-1)
        gif = jnp.dot(uf, wih1f, preferred_element_type=f32) + bih1f
        ghf = jnp.dot(hf, whh1f, preferred_element_type=f32) + bhh1f
        hf = _gru_gate(gif, ghf, hf)
        gib = jnp.dot(ub, wih1b, preferred_element_type=f32) + bih1b
        ghb = jnp.dot(hb, whh1b, preferred_element_type=f32) + bhh1b
        hb = _gru_gate(gib, ghb, hb)
        h2f_ref[t] = hf
        h2b_ref[tb] = hb
        return (hf, hb)

    jax.lax.fori_loop(0, _S, l1_step, (h0, h0))

    h2 = jnp.concatenate([h2f_ref[...], h2b_ref[...]], axis=-1)
    h2flat = jnp.reshape(h2, (_S * _B, 2 * _H))
    a = _gelu(jnp.dot(h2flat, aW1_ref[0], preferred_element_type=f32)
              + ab1_ref[0])
    sflat = jnp.sum(a * aW2_ref[0], axis=-1, keepdims=True) + ab2_ref[0]
    s3 = jnp.reshape(sflat, (_S, _B, 1))
    m = jnp.max(s3, axis=0, keepdims=True)
    ex = jnp.exp(s3 - m)
    att = ex / jnp.sum(ex, axis=0, keepdims=True)
    ctx = jnp.sum(h2 * att, axis=0)
    t1 = jnp.dot(ctx, pW1_ref[0], preferred_element_type=f32) + pb1_ref[0]
    mu = jnp.mean(t1, axis=-1, keepdims=True)
    var = jnp.mean((t1 - mu) ** 2, axis=-1, keepdims=True)
    t2 = (t1 - mu) / jnp.sqrt(var + 1e-5) * lng_ref[0] + lnb_ref[0]
    g = _gelu(t2)
    eo_ref[0] = jnp.sum(g * pW2_ref[0], axis=-1, keepdims=True) + pb2_ref[0]


def _router_body(xs_ref, dw1_ref, dwb1_ref, pw1_ref, pwb1_ref, bn1s_ref,
                 bn1b_ref, dw2_ref, dwb2_ref, pw2_ref, pwb2_ref, bn2s_ref,
                 bn2b_ref, gcW_ref, gcb_ref, gW1_ref, gb1_ref, gW2_ref,
                 gb2_ref, temp_ref, rw_ref):
    f32 = _F32
    x = xs_ref[...]
    zrow = jnp.zeros((1, _B, _C), f32)
    xm = jnp.concatenate([zrow, x[:-1]], axis=0)
    xp = jnp.concatenate([x[1:], zrow], axis=0)
    h = xm * dw1_ref[0] + x * dw1_ref[1] + xp * dw1_ref[2] + dwb1_ref[0]
    hf = jnp.reshape(h, (_S * _B, _C))
    hf = jnp.dot(hf, pw1_ref[...], preferred_element_type=f32) + pwb1_ref[...]
    hf = _gelu(hf) * bn1s_ref[...] + bn1b_ref[...]
    h = jnp.reshape(hf, (_S, _B, _RH))
    zrow2 = jnp.zeros((1, _B, _RH), f32)
    hm = jnp.concatenate([zrow2, h[:-1]], axis=0)
    hp = jnp.concatenate([h[1:], zrow2], axis=0)
    h = hm * dw2_ref[0] + h * dw2_ref[1] + hp * dw2_ref[2] + dwb2_ref[0]
    hf = jnp.reshape(h, (_S * _B, _RH))
    hf = jnp.dot(hf, pw2_ref[...], preferred_element_type=f32) + pwb2_ref[...]
    hf = _gelu(hf) * bn2s_ref[...] + bn2b_ref[...]
    h = jnp.reshape(hf, (_S, _B, _RH))
    mean_h = jnp.mean(h, axis=0)
    g2 = _gelu(jnp.dot(mean_h, gcW_ref[...], preferred_element_type=f32)
               + gcb_ref[...])
    comb = jnp.concatenate([mean_h, g2], axis=-1)
    z1 = _gelu(jnp.dot(comb, gW1_ref[...], preferred_element_type=f32)
               + gb1_ref[...])
    logits = jnp.dot(z1, gW2_ref[...], preferred_element_type=f32) + gb2_ref[...]
    logits = logits / (jnp.abs(temp_ref[0, 0]) + 1e-7)
    lm = jnp.max(logits, axis=-1, keepdims=True)
    le = jnp.exp(logits - lm)
    rw_ref[...] = le / jnp.sum(le, axis=-1, keepdims=True)


def _combine_body(rw_ref, eo_ref, out_ref):
    rw = rw_ref[...]
    eo = eo_ref[...]
    neg = jnp.float32(-3.0e38)
    idx = jax.lax.broadcasted_iota(jnp.int32, (_B, _NE), 1)
    v1 = jnp.max(rw, axis=-1, keepdims=True)
    i1 = jnp.min(jnp.where(rw >= v1, idx, _NE), axis=-1, keepdims=True)
    m1 = idx == i1
    rw2 = jnp.where(m1, neg, rw)
    v2 = jnp.max(rw2, axis=-1, keepdims=True)
    i2 = jnp.min(jnp.where(rw2 >= v2, idx, _NE), axis=-1, keepdims=True)
    sel1 = jnp.sum(jnp.where(m1, eo, 0.0), axis=-1, keepdims=True)
    sel2 = jnp.sum(jnp.where(idx == i2, eo, 0.0), axis=-1, keepdims=True)
    wsum = jnp.maximum(jnp.abs(v1) + jnp.abs(v2), 1e-12)
    out_ref[...] = (v1 * sel1 + v2 * sel2) / wsum


def kernel(x, gWih0, gWhh0, gbih0, gbhh0, gWih1, gWhh1, gbih1, gbhh1,
           attW1, attb1, attW2, attb2, projW1, projb1, ln_g, ln_b,
           projW2, projb2, dw1, dwb1, pw1, pwb1, bn1g, bn1b, dw2, dwb2,
           pw2, pwb2, bn2g, bn2b, gcW, gcb, gateW1, gateb1, gateW2,
           gateb2, temp):
    f32 = _F32
    xs = jnp.transpose(x, (1, 0, 2))

    wih0T = jnp.transpose(gWih0, (0, 1, 3, 2))
    whh0T = jnp.transpose(gWhh0, (0, 1, 3, 2))
    wih1T = jnp.transpose(gWih1, (0, 1, 3, 2))
    whh1T = jnp.transpose(gWhh1, (0, 1, 3, 2))
    bih0r = gbih0[:, :, None, :]
    bhh0r = gbhh0[:, :, None, :]
    bih1r = gbih1[:, :, None, :]
    bhh1r = gbhh1[:, :, None, :]
    attW1T = jnp.transpose(attW1, (0, 2, 1))
    attb1r = attb1[:, None, :]
    attb2r = attb2[:, :, None]
    projW1T = jnp.transpose(projW1, (0, 2, 1))
    projb1r = projb1[:, None, :]
    lngr = ln_g[:, None, :]
    lnbr = ln_b[:, None, :]
    projb2r = projb2[:, :, None]

    full = lambda shp: pl.BlockSpec(shp, lambda e: (0,) * len(shp))
    per_e = lambda shp: pl.BlockSpec((1,) + shp, lambda e: (e,) + (0,) * len(shp))
    eo = pl.pallas_call(
        _expert_body,
        grid=(_NE,),
        in_specs=[
            full((_S, _B, _C)),
            per_e((2, _C, 3 * _H)), per_e((2, _H, 3 * _H)),
            per_e((2, 1, 3 * _H)), per_e((2, 1, 3 * _H)),
            per_e((2, 2 * _H, 3 * _H)), per_e((2, _H, 3 * _H)),
            per_e((2, 1, 3 * _H)), per_e((2, 1, 3 * _H)),
            per_e((2 * _H, _H)), per_e((1, _H)), per_e((1, _H)),
            per_e((1, 1)),
            per_e((2 * _H, _H)), per_e((1, _H)), per_e((1, _H)),
            per_e((1, _H)), per_e((1, _H)), per_e((1, 1)),
        ],
        out_specs=pl.BlockSpec((1, _B, 1), lambda e: (e, 0, 0)),
        out_shape=jax.ShapeDtypeStruct((_NE, _B, 1), f32),
        scratch_shapes=[
            pltpu.VMEM((_S, _B, _H), f32), pltpu.VMEM((_S, _B, _H), f32),
            pltpu.VMEM((_S, _B, _H), f32), pltpu.VMEM((_S, _B, _H), f32),
        ],
    )(xs, wih0T, whh0T, bih0r, bhh0r, wih1T, whh1T, bih1r, bhh1r,
      attW1T, attb1r, attW2, attb2r, projW1T, projb1r, lngr, lnbr,
      projW2, projb2r)

    dw1k = jnp.transpose(dw1[:, 0, :])[:, None, None, :]
    dwb1r = dwb1[None, None, None, :]
    pw1T = jnp.transpose(pw1[:, :, 0])
    pwb1r = pwb1[None, :]
    bn1s = (bn1g / jnp.sqrt(1.0 + 1e-5))[None, :]
    bn1br = bn1b[None, :]
    dw2k = jnp.transpose(dw2[:, 0, :])[:, None, None, :]
    dwb2r = dwb2[None, None, None, :]
    pw2T = jnp.transpose(pw2[:, :, 0])
    pwb2r = pwb2[None, :]
    bn2s = (bn2g / jnp.sqrt(1.0 + 1e-5))[None, :]
    bn2br = bn2b[None, :]
    gcWT = jnp.transpose(gcW[:, :, 0])
    gcbr = gcb[None, :]
    gW1T = jnp.transpose(gateW1)
    gb1r = gateb1[None, :]
    gW2T = jnp.transpose(gateW2)
    gb2r = gateb2[None, :]
    temp2 = jnp.reshape(temp, (1, 1))

    rw = pl.pallas_call(
        _router_body,
        out_shape=jax.ShapeDtypeStruct((_B, _NE), f32),
    )(xs, dw1k, dwb1r, pw1T, pwb1r, bn1s, bn1br, dw2k, dwb2r, pw2T,
      pwb2r, bn2s, bn2br, gcWT, gcbr, gW1T, gb1r, gW2T, gb2r, temp2)

    eoT = jnp.transpose(eo[:, :, 0])
    out = pl.pallas_call(
        _combine_body,
        out_shape=jax.ShapeDtypeStruct((_B, 1), f32),
    )(rw, eoT)
    return out


# speedup vs baseline: 10.7142x; 1.4660x over previous
import jax
import jax.numpy as jnp
from jax.experimental import pallas as pl
from jax.experimental.pallas import tpu as pltpu

_B, _S, _C = 64, 64, 64
_H, _NE, _RH = 128, 4, 64
_F32 = jnp.float32


def _gelu(x):
    return 0.5 * x * (1.0 + jax.lax.erf(x * 0.7071067811865476))


def _gru_gate(gi, gh, h):
    r = jax.nn.sigmoid(gi[:, :_H] + gh[:, :_H])
    z = jax.nn.sigmoid(gi[:, _H:2 * _H] + gh[:, _H:2 * _H])
    n = jnp.tanh(gi[:, 2 * _H:] + r * gh[:, 2 * _H:])
    return (1.0 - z) * n + z * h


def _mega_body(xs_ref, w0f_ref, w0b_ref, bih0_ref, whh0_ref, bhh0_ref,
               wih1_ref, bih1_ref, whh1_ref, bhh1_ref,
               aW1_ref, ab1_ref, aW2_ref, ab2_ref,
               pW1_ref, pb1_ref, lng_ref, lnb_ref, pW2_ref, pb2_ref,
               dw1_ref, dwb1_ref, pw1_ref, pwb1_ref, bn1s_ref, bn1b_ref,
               dw2_ref, dwb2_ref, pw2_ref, pwb2_ref, bn2s_ref, bn2b_ref,
               gcW_ref, gcb_ref, gW1_ref, gb1_ref, gW2_ref, gb2_ref,
               temp_ref,
               eo_ref, rw_ref, ysf_ref, ysb_ref, h2f_ref, h2b_ref):
    f32 = _F32
    h0 = jnp.zeros((_B, _H), f32)
    G = 3 * _H

    def l0_step(t, hs):
        hf, hb = hs
        xt = xs_ref[t]
        xb = xs_ref[_S - 1 - t]
        gif_all = jnp.dot(xt, w0f_ref[...], preferred_element_type=f32)
        gib_all = jnp.dot(xb, w0b_ref[...], preferred_element_type=f32)
        nhf, nhb = [], []
        for e in range(_NE):
            gif = gif_all[:, e * G:(e + 1) * G] + bih0_ref[e, 0]
            ghf = jnp.dot(hf[e], whh0_ref[e, 0],
                          preferred_element_type=f32) + bhh0_ref[e, 0]
            hfe = _gru_gate(gif, ghf, hf[e])
            gib = gib_all[:, e * G:(e + 1) * G] + bih0_ref[e, 1]
            ghb = jnp.dot(hb[e], whh0_ref[e, 1],
                          preferred_element_type=f32) + bhh0_ref[e, 1]
            hbe = _gru_gate(gib, ghb, hb[e])
            ysf_ref[e, t] = hfe
            ysb_ref[e, _S - 1 - t] = hbe
            nhf.append(hfe)
            nhb.append(hbe)
        return (tuple(nhf), tuple(nhb))

    jax.lax.fori_loop(0, _S, l0_step, ((h0,) * _NE, (h0,) * _NE))

    def l1_step(t, hs):
        hf, hb = hs
        tb = _S - 1 - t
        nhf, nhb = [], []
        for e in range(_NE):
            uf = jnp.concatenate([ysf_ref[e, t], ysb_ref[e, t]], axis=-1)
            ub = jnp.concatenate([ysf_ref[e, tb], ysb_ref[e, tb]], axis=-1)
            gif = jnp.dot(uf, wih1_ref[e, 0],
                          preferred_element_type=f32) + bih1_ref[e, 0]
            ghf = jnp.dot(hf[e], whh1_ref[e, 0],
                          preferred_element_type=f32) + bhh1_ref[e, 0]
            hfe = _gru_gate(gif, ghf, hf[e])
            gib = jnp.dot(ub, wih1_ref[e, 1],
                          preferred_element_type=f32) + bih1_ref[e, 1]
            ghb = jnp.dot(hb[e], whh1_ref[e, 1],
                          preferred_element_type=f32) + bhh1_ref[e, 1]
            hbe = _gru_gate(gib, ghb, hb[e])
            h2f_ref[e, t] = hfe
            h2b_ref[e, tb] = hbe
            nhf.append(hfe)
            nhb.append(hbe)
        return (tuple(nhf), tuple(nhb))

    jax.lax.fori_loop(0, _S, l1_step, ((h0,) * _NE, (h0,) * _NE))

    for e in range(_NE):
        h2 = jnp.concatenate([h2f_ref[e], h2b_ref[e]], axis=-1)
        h2flat = jnp.reshape(h2, (_S * _B, 2 * _H))
        a = _gelu(jnp.dot(h2flat, aW1_ref[e], preferred_element_type=f32)
                  + ab1_ref[e])
        sflat = jnp.sum(a * aW2_ref[e], axis=-1, keepdims=True) + ab2_ref[e]
        s3 = jnp.reshape(sflat, (_S, _B, 1))
        m = jnp.max(s3, axis=0, keepdims=True)
        ex = jnp.exp(s3 - m)
        att = ex / jnp.sum(ex, axis=0, keepdims=True)
        ctx = jnp.sum(h2 * att, axis=0)
        t1 = jnp.dot(ctx, pW1_ref[e], preferred_element_type=f32) + pb1_ref[e]
        mu = jnp.mean(t1, axis=-1, keepdims=True)
        var = jnp.mean((t1 - mu) ** 2, axis=-1, keepdims=True)
        t2 = (t1 - mu) / jnp.sqrt(var + 1e-5) * lng_ref[e] + lnb_ref[e]
        g = _gelu(t2)
        eo_ref[e] = jnp.sum(g * pW2_ref[e], axis=-1, keepdims=True) + pb2_ref[e]

    x = xs_ref[...]
    zrow = jnp.zeros((1, _B, _C), f32)
    xm = jnp.concatenate([zrow, x[:-1]], axis=0)
    xp = jnp.concatenate([x[1:], zrow], axis=0)
    h = xm * dw1_ref[0] + x * dw1_ref[1] + xp * dw1_ref[2] + dwb1_ref[0]
    hfl = jnp.reshape(h, (_S * _B, _C))
    hfl = jnp.dot(hfl, pw1_ref[...], preferred_element_type=f32) + pwb1_ref[...]
    hfl = _gelu(hfl) * bn1s_ref[...] + bn1b_ref[...]
    h = jnp.reshape(hfl, (_S, _B, _RH))
    zrow2 = jnp.zeros((1, _B, _RH), f32)
    hm = jnp.concatenate([zrow2, h[:-1]], axis=0)
    hp = jnp.concatenate([h[1:], zrow2], axis=0)
    h = hm * dw2_ref[0] + h * dw2_ref[1] + hp * dw2_ref[2] + dwb2_ref[0]
    hfl = jnp.reshape(h, (_S * _B, _RH))
    hfl = jnp.dot(hfl, pw2_ref[...], preferred_element_type=f32) + pwb2_ref[...]
    hfl = _gelu(hfl) * bn2s_ref[...] + bn2b_ref[...]
    h = jnp.reshape(hfl, (_S, _B, _RH))
    mean_h = jnp.mean(h, axis=0)
    g2 = _gelu(jnp.dot(mean_h, gcW_ref[...], preferred_element_type=f32)
               + gcb_ref[...])
    comb = jnp.concatenate([mean_h, g2], axis=-1)
    z1 = _gelu(jnp.dot(comb, gW1_ref[...], preferred_element_type=f32)
               + gb1_ref[...])
    logits = jnp.dot(z1, gW2_ref[...], preferred_element_type=f32) + gb2_ref[...]
    logits = logits / (jnp.abs(temp_ref[0, 0]) + 1e-7)
    lm = jnp.max(logits, axis=-1, keepdims=True)
    le = jnp.exp(logits - lm)
    rw_ref[...] = le / jnp.sum(le, axis=-1, keepdims=True)


def _combine_body(rw_ref, eo_ref, out_ref):
    rw = rw_ref[...]
    eo = eo_ref[...]
    neg = jnp.float32(-3.0e38)
    idx = jax.lax.broadcasted_iota(jnp.int32, (_B, _NE), 1)
    v1 = jnp.max(rw, axis=-1, keepdims=True)
    i1 = jnp.min(jnp.where(rw >= v1, idx, _NE), axis=-1, keepdims=True)
    m1 = idx == i1
    rw2 = jnp.where(m1, neg, rw)
    v2 = jnp.max(rw2, axis=-1, keepdims=True)
    i2 = jnp.min(jnp.where(rw2 >= v2, idx, _NE), axis=-1, keepdims=True)
    sel1 = jnp.sum(jnp.where(m1, eo, 0.0), axis=-1, keepdims=True)
    sel2 = jnp.sum(jnp.where(idx == i2, eo, 0.0), axis=-1, keepdims=True)
    wsum = jnp.maximum(jnp.abs(v1) + jnp.abs(v2), 1e-12)
    out_ref[...] = (v1 * sel1 + v2 * sel2) / wsum


def kernel(x, gWih0, gWhh0, gbih0, gbhh0, gWih1, gWhh1, gbih1, gbhh1,
           attW1, attb1, attW2, attb2, projW1, projb1, ln_g, ln_b,
           projW2, projb2, dw1, dwb1, pw1, pwb1, bn1g, bn1b, dw2, dwb2,
           pw2, pwb2, bn2g, bn2b, gcW, gcb, gateW1, gateb1, gateW2,
           gateb2, temp):
    f32 = _F32
    xs = jnp.transpose(x, (1, 0, 2))

    wih0T = jnp.transpose(gWih0, (0, 1, 3, 2))
    w0f = jnp.reshape(jnp.transpose(wih0T[:, 0], (1, 0, 2)), (_C, _NE * 3 * _H))
    w0b = jnp.reshape(jnp.transpose(wih0T[:, 1], (1, 0, 2)), (_C, _NE * 3 * _H))
    whh0T = jnp.transpose(gWhh0, (0, 1, 3, 2))
    wih1T = jnp.transpose(gWih1, (0, 1, 3, 2))
    whh1T = jnp.transpose(gWhh1, (0, 1, 3, 2))
    bih0r = gbih0[:, :, None, :]
    bhh0r = gbhh0[:, :, None, :]
    bih1r = gbih1[:, :, None, :]
    bhh1r = gbhh1[:, :, None, :]
    attW1T = jnp.transpose(attW1, (0, 2, 1))
    attb1r = attb1[:, None, :]
    attb2r = attb2[:, :, None]
    projW1T = jnp.transpose(projW1, (0, 2, 1))
    projb1r = projb1[:, None, :]
    lngr = ln_g[:, None, :]
    lnbr = ln_b[:, None, :]
    projb2r = projb2[:, :, None]

    dw1k = jnp.transpose(dw1[:, 0, :])[:, None, None, :]
    dwb1r = dwb1[None, None, None, :]
    pw1T = jnp.transpose(pw1[:, :, 0])
    pwb1r = pwb1[None, :]
    bn1s = (bn1g / jnp.sqrt(1.0 + 1e-5))[None, :]
    bn1br = bn1b[None, :]
    dw2k = jnp.transpose(dw2[:, 0, :])[:, None, None, :]
    dwb2r = dwb2[None, None, None, :]
    pw2T = jnp.transpose(pw2[:, :, 0])
    pwb2r = pwb2[None, :]
    bn2s = (bn2g / jnp.sqrt(1.0 + 1e-5))[None, :]
    bn2br = bn2b[None, :]
    gcWT = jnp.transpose(gcW[:, :, 0])
    gcbr = gcb[None, :]
    gW1T = jnp.transpose(gateW1)
    gb1r = gateb1[None, :]
    gW2T = jnp.transpose(gateW2)
    gb2r = gateb2[None, :]
    temp2 = jnp.reshape(temp, (1, 1))

    eo, rw = pl.pallas_call(
        _mega_body,
        out_shape=(jax.ShapeDtypeStruct((_NE, _B, 1), f32),
                   jax.ShapeDtypeStruct((_B, _NE), f32)),
        scratch_shapes=[
            pltpu.VMEM((_NE, _S, _B, _H), f32),
            pltpu.VMEM((_NE, _S, _B, _H), f32),
            pltpu.VMEM((_NE, _S, _B, _H), f32),
            pltpu.VMEM((_NE, _S, _B, _H), f32),
        ],
    )(xs, w0f, w0b, bih0r, whh0T, bhh0r, wih1T, bih1r, whh1T, bhh1r,
      attW1T, attb1r, attW2, attb2r, projW1T, projb1r, lngr, lnbr,
      projW2, projb2r,
      dw1k, dwb1r, pw1T, pwb1r, bn1s, bn1br, dw2k, dwb2r, pw2T, pwb2r,
      bn2s, bn2br, gcWT, gcbr, gW1T, gb1r, gW2T, gb2r, temp2)

    eoT = jnp.transpose(eo[:, :, 0])
    out = pl.pallas_call(
        _combine_body,
        out_shape=jax.ShapeDtypeStruct((_B, 1), f32),
    )(rw, eoT)
    return out


# speedup vs baseline: 10.9529x; 1.0223x over previous
import jax
import jax.numpy as jnp
from jax.experimental import pallas as pl
from jax.experimental.pallas import tpu as pltpu

_B, _S, _C = 64, 64, 64
_H, _NE, _RH = 128, 4, 64
_F32 = jnp.float32


def _gelu(x):
    return 0.5 * x * (1.0 + jax.lax.erf(x * 0.7071067811865476))


def _gru_gate(gi, gh, h):
    r = jax.nn.sigmoid(gi[:, :_H] + gh[:, :_H])
    z = jax.nn.sigmoid(gi[:, _H:2 * _H] + gh[:, _H:2 * _H])
    n = jnp.tanh(gi[:, 2 * _H:] + r * gh[:, 2 * _H:])
    return (1.0 - z) * n + z * h


def _mega_body(xs_ref, w0f_ref, w0b_ref, bih0_ref, whh0_ref, bhh0_ref,
               wih1_ref, bih1_ref, whh1_ref, bhh1_ref,
               aW1_ref, ab1_ref, aW2_ref, ab2_ref,
               pW1_ref, pb1_ref, lng_ref, lnb_ref, pW2_ref, pb2_ref,
               dw1_ref, dwb1_ref, pw1_ref, pwb1_ref, bn1s_ref, bn1b_ref,
               dw2_ref, dwb2_ref, pw2_ref, pwb2_ref, bn2s_ref, bn2b_ref,
               gcW_ref, gcb_ref, gW1_ref, gb1_ref, gW2_ref, gb2_ref,
               temp_ref,
               eo_ref, rw_ref, ysf_ref, ysb_ref, h2f_ref, h2b_ref):
    f32 = _F32
    bf16 = jnp.bfloat16
    h0 = jnp.zeros((_B, _H), f32)
    G = 3 * _H

    def l0_step(t, hs):
        hf, hb = hs
        xt = xs_ref[t].astype(bf16)
        xb = xs_ref[_S - 1 - t].astype(bf16)
        gif_all = jnp.dot(xt, w0f_ref[...], preferred_element_type=f32)
        gib_all = jnp.dot(xb, w0b_ref[...], preferred_element_type=f32)
        nhf, nhb = [], []
        for e in range(_NE):
            gif = gif_all[:, e * G:(e + 1) * G] + bih0_ref[e, 0]
            ghf = jnp.dot(hf[e].astype(bf16), whh0_ref[e, 0],
                          preferred_element_type=f32) + bhh0_ref[e, 0]
            hfe = _gru_gate(gif, ghf, hf[e])
            gib = gib_all[:, e * G:(e + 1) * G] + bih0_ref[e, 1]
            ghb = jnp.dot(hb[e].astype(bf16), whh0_ref[e, 1],
                          preferred_element_type=f32) + bhh0_ref[e, 1]
            hbe = _gru_gate(gib, ghb, hb[e])
            ysf_ref[e, t] = hfe.astype(bf16)
            ysb_ref[e, _S - 1 - t] = hbe.astype(bf16)
            nhf.append(hfe)
            nhb.append(hbe)
        return (tuple(nhf), tuple(nhb))

    jax.lax.fori_loop(0, _S, l0_step, ((h0,) * _NE, (h0,) * _NE))

    def l1_step(t, hs):
        hf, hb = hs
        tb = _S - 1 - t
        nhf, nhb = [], []
        for e in range(_NE):
            uf = jnp.concatenate([ysf_ref[e, t], ysb_ref[e, t]], axis=-1)
            ub = jnp.concatenate([ysf_ref[e, tb], ysb_ref[e, tb]], axis=-1)
            gif = jnp.dot(uf, wih1_ref[e, 0],
                          preferred_element_type=f32) + bih1_ref[e, 0]
            ghf = jnp.dot(hf[e].astype(bf16), whh1_ref[e, 0],
                          preferred_element_type=f32) + bhh1_ref[e, 0]
            hfe = _gru_gate(gif, ghf, hf[e])
            gib = jnp.dot(ub, wih1_ref[e, 1],
                          preferred_element_type=f32) + bih1_ref[e, 1]
            ghb = jnp.dot(hb[e].astype(bf16), whh1_ref[e, 1],
                          preferred_element_type=f32) + bhh1_ref[e, 1]
            hbe = _gru_gate(gib, ghb, hb[e])
            h2f_ref[e, t] = hfe.astype(bf16)
            h2b_ref[e, tb] = hbe.astype(bf16)
            nhf.append(hfe)
            nhb.append(hbe)
        return (tuple(nhf), tuple(nhb))

    jax.lax.fori_loop(0, _S, l1_step, ((h0,) * _NE, (h0,) * _NE))

    for e in range(_NE):
        h2b16 = jnp.concatenate([h2f_ref[e], h2b_ref[e]], axis=-1)
        h2 = h2b16.astype(f32)
        h2flat = jnp.reshape(h2b16, (_S * _B, 2 * _H))
        a = _gelu(jnp.dot(h2flat, aW1_ref[e], preferred_element_type=f32)
                  + ab1_ref[e])
        sflat = jnp.sum(a * aW2_ref[e], axis=-1, keepdims=True) + ab2_ref[e]
        s3 = jnp.reshape(sflat, (_S, _B, 1))
        m = jnp.max(s3, axis=0, keepdims=True)
        ex = jnp.exp(s3 - m)
        att = ex / jnp.sum(ex, axis=0, keepdims=True)
        ctx = jnp.sum(h2 * att, axis=0)
        t1 = jnp.dot(ctx, pW1_ref[e], preferred_element_type=f32) + pb1_ref[e]
        mu = jnp.mean(t1, axis=-1, keepdims=True)
        var = jnp.mean((t1 - mu) ** 2, axis=-1, keepdims=True)
        t2 = (t1 - mu) / jnp.sqrt(var + 1e-5) * lng_ref[e] + lnb_ref[e]
        g = _gelu(t2)
        eo_ref[e] = jnp.sum(g * pW2_ref[e], axis=-1, keepdims=True) + pb2_ref[e]

    x = xs_ref[...]
    zrow = jnp.zeros((1, _B, _C), f32)
    xm = jnp.concatenate([zrow, x[:-1]], axis=0)
    xp = jnp.concatenate([x[1:], zrow], axis=0)
    h = xm * dw1_ref[0] + x * dw1_ref[1] + xp * dw1_ref[2] + dwb1_ref[0]
    hfl = jnp.reshape(h, (_S * _B, _C))
    hfl = jnp.dot(hfl, pw1_ref[...], preferred_element_type=f32) + pwb1_ref[...]
    hfl = _gelu(hfl) * bn1s_ref[...] + bn1b_ref[...]
    h = jnp.reshape(hfl, (_S, _B, _RH))
    zrow2 = jnp.zeros((1, _B, _RH), f32)
    hm = jnp.concatenate([zrow2, h[:-1]], axis=0)
    hp = jnp.concatenate([h[1:], zrow2], axis=0)
    h = hm * dw2_ref[0] + h * dw2_ref[1] + hp * dw2_ref[2] + dwb2_ref[0]
    hfl = jnp.reshape(h, (_S * _B, _RH))
    hfl = jnp.dot(hfl, pw2_ref[...], preferred_element_type=f32) + pwb2_ref[...]
    hfl = _gelu(hfl) * bn2s_ref[...] + bn2b_ref[...]
    h = jnp.reshape(hfl, (_S, _B, _RH))
    mean_h = jnp.mean(h, axis=0)
    g2 = _gelu(jnp.dot(mean_h, gcW_ref[...], preferred_element_type=f32)
               + gcb_ref[...])
    comb = jnp.concatenate([mean_h, g2], axis=-1)
    z1 = _gelu(jnp.dot(comb, gW1_ref[...], preferred_element_type=f32)
               + gb1_ref[...])
    logits = jnp.dot(z1, gW2_ref[...], preferred_element_type=f32) + gb2_ref[...]
    logits = logits / (jnp.abs(temp_ref[0, 0]) + 1e-7)
    lm = jnp.max(logits, axis=-1, keepdims=True)
    le = jnp.exp(logits - lm)
    rw_ref[...] = le / jnp.sum(le, axis=-1, keepdims=True)


def _combine_body(rw_ref, eo_ref, out_ref):
    rw = rw_ref[...]
    eo = eo_ref[...]
    neg = jnp.float32(-3.0e38)
    idx = jax.lax.broadcasted_iota(jnp.int32, (_B, _NE), 1)
    v1 = jnp.max(rw, axis=-1, keepdims=True)
    i1 = jnp.min(jnp.where(rw >= v1, idx, _NE), axis=-1, keepdims=True)
    m1 = idx == i1
    rw2 = jnp.where(m1, neg, rw)
    v2 = jnp.max(rw2, axis=-1, keepdims=True)
    i2 = jnp.min(jnp.where(rw2 >= v2, idx, _NE), axis=-1, keepdims=True)
    sel1 = jnp.sum(jnp.where(m1, eo, 0.0), axis=-1, keepdims=True)
    sel2 = jnp.sum(jnp.where(idx == i2, eo, 0.0), axis=-1, keepdims=True)
    wsum = jnp.maximum(jnp.abs(v1) + jnp.abs(v2), 1e-12)
    out_ref[...] = (v1 * sel1 + v2 * sel2) / wsum


def kernel(x, gWih0, gWhh0, gbih0, gbhh0, gWih1, gWhh1, gbih1, gbhh1,
           attW1, attb1, attW2, attb2, projW1, projb1, ln_g, ln_b,
           projW2, projb2, dw1, dwb1, pw1, pwb1, bn1g, bn1b, dw2, dwb2,
           pw2, pwb2, bn2g, bn2b, gcW, gcb, gateW1, gateb1, gateW2,
           gateb2, temp):
    f32 = _F32
    xs = jnp.transpose(x, (1, 0, 2))

    bf16 = jnp.bfloat16
    wih0T = jnp.transpose(gWih0, (0, 1, 3, 2))
    w0f = jnp.reshape(jnp.transpose(wih0T[:, 0], (1, 0, 2)),
                      (_C, _NE * 3 * _H)).astype(bf16)
    w0b = jnp.reshape(jnp.transpose(wih0T[:, 1], (1, 0, 2)),
                      (_C, _NE * 3 * _H)).astype(bf16)
    whh0T = jnp.transpose(gWhh0, (0, 1, 3, 2)).astype(bf16)
    wih1T = jnp.transpose(gWih1, (0, 1, 3, 2)).astype(bf16)
    whh1T = jnp.transpose(gWhh1, (0, 1, 3, 2)).astype(bf16)
    bih0r = gbih0[:, :, None, :]
    bhh0r = gbhh0[:, :, None, :]
    bih1r = gbih1[:, :, None, :]
    bhh1r = gbhh1[:, :, None, :]
    attW1T = jnp.transpose(attW1, (0, 2, 1)).astype(bf16)
    attb1r = attb1[:, None, :]
    attb2r = attb2[:, :, None]
    projW1T = jnp.transpose(projW1, (0, 2, 1))
    projb1r = projb1[:, None, :]
    lngr = ln_g[:, None, :]
    lnbr = ln_b[:, None, :]
    projb2r = projb2[:, :, None]

    dw1k = jnp.transpose(dw1[:, 0, :])[:, None, None, :]
    dwb1r = dwb1[None, None, None, :]
    pw1T = jnp.transpose(pw1[:, :, 0])
    pwb1r = pwb1[None, :]
    bn1s = (bn1g / jnp.sqrt(1.0 + 1e-5))[None, :]
    bn1br = bn1b[None, :]
    dw2k = jnp.transpose(dw2[:, 0, :])[:, None, None, :]
    dwb2r = dwb2[None, None, None, :]
    pw2T = jnp.transpose(pw2[:, :, 0])
    pwb2r = pwb2[None, :]
    bn2s = (bn2g / jnp.sqrt(1.0 + 1e-5))[None, :]
    bn2br = bn2b[None, :]
    gcWT = jnp.transpose(gcW[:, :, 0])
    gcbr = gcb[None, :]
    gW1T = jnp.transpose(gateW1)
    gb1r = gateb1[None, :]
    gW2T = jnp.transpose(gateW2)
    gb2r = gateb2[None, :]
    temp2 = jnp.reshape(temp, (1, 1))

    eo, rw = pl.pallas_call(
        _mega_body,
        out_shape=(jax.ShapeDtypeStruct((_NE, _B, 1), f32),
                   jax.ShapeDtypeStruct((_B, _NE), f32)),
        scratch_shapes=[
            pltpu.VMEM((_NE, _S, _B, _H), bf16),
            pltpu.VMEM((_NE, _S, _B, _H), bf16),
            pltpu.VMEM((_NE, _S, _B, _H), bf16),
            pltpu.VMEM((_NE, _S, _B, _H), bf16),
        ],
    )(xs, w0f, w0b, bih0r, whh0T, bhh0r, wih1T, bih1r, whh1T, bhh1r,
      attW1T, attb1r, attW2, attb2r, projW1T, projb1r, lngr, lnbr,
      projW2, projb2r,
      dw1k, dwb1r, pw1T, pwb1r, bn1s, bn1br, dw2k, dwb2r, pw2T, pwb2r,
      bn2s, bn2br, gcWT, gcbr, gW1T, gb1r, gW2T, gb2r, temp2)

    eoT = jnp.transpose(eo[:, :, 0])
    out = pl.pallas_call(
        _combine_body,
        out_shape=jax.ShapeDtypeStruct((_B, 1), f32),
    )(rw, eoT)
    return out
